# baseline (speedup 1.0000x reference)
import functools

import jax
import jax.numpy as jnp
from jax import lax
from jax.experimental import pallas as pl
from jax.experimental.pallas import tpu as pltpu
from jax.experimental.pallas import tpu_sc as plsc

_B, _T, _K, _D = 64, 4096, 512, 8
_NC, _NS, _L = 2, 16, 16
_NW = _NC * _NS
_BPW = _B // _NW
_CHUNKS = 4


def _nn_body(w_ref, nn_ref):
    w = w_ref[...]
    w2_col = jnp.sum(w * w, axis=1, keepdims=True)
    ones = jnp.ones((1, _D), jnp.float32)
    w2_row = lax.dot_general(ones, w * w,
                             (((1,), (1,)), ((), ())),
                             preferred_element_type=jnp.float32)
    g = lax.dot_general(w, w, (((1,), (1,)), ((), ())),
                        preferred_element_type=jnp.float32)
    d2 = jnp.maximum(w2_col - 2.0 * g + w2_row, 0.0)
    dist = jnp.sqrt(d2)
    mn = jnp.min(dist, axis=1, keepdims=True)
    col = lax.broadcasted_iota(jnp.int32, (_K, _K), 1)
    nn = jnp.min(jnp.where(dist == mn, col, _K), axis=1)
    nn_ref[...] = nn


_nn_call = pl.pallas_call(
    _nn_body,
    out_shape=jax.ShapeDtypeStruct((_K,), jnp.int32),
)


_sc_mesh = plsc.VectorSubcoreMesh(core_axis_name="c", subcore_axis_name="s")


@functools.partial(
    pl.kernel,
    mesh=_sc_mesh,
    compiler_params=pltpu.CompilerParams(needs_layout_passes=False,
                                         disable_bounds_checks=True),
    out_type=[
        jax.ShapeDtypeStruct((_B, _D, _T), jnp.float32),
        jax.ShapeDtypeStruct((_B, _T), jnp.int32),
    ],
    scratch_types=[
        pltpu.VMEM((_K * _D,), jnp.float32),
        pltpu.VMEM((_K,), jnp.int32),
        pltpu.VMEM((_BPW, _T), jnp.int32),
        pltpu.VMEM((_BPW, _D, _T), jnp.float32),
        pltpu.VMEM((_BPW, _T), jnp.int32),
        pltpu.SemaphoreType.DMA,
        pltpu.SemaphoreType.DMA,
        pltpu.SemaphoreType.DMA,
    ],
)
def _sc_gather(w_hbm, nn_hbm, codes_hbm, lat_hbm, ch_hbm,
               w_v, nn_v, codes_v, lat_v, ch_v, sem_c0, sem_c1, sem_o):
    wid = lax.axis_index("s") * _NC + lax.axis_index("c")
    b0 = wid * _BPW
    sems = [sem_c0, sem_c1]
    in_cps = [pltpu.async_copy(codes_hbm.at[b0 + i], codes_v.at[i], sems[i])
              for i in range(_BPW)]
    pltpu.sync_copy(w_hbm, w_v)
    pltpu.sync_copy(nn_hbm, nn_v)
    out_cps = []
    tc = _T // _CHUNKS
    for i in range(_BPW):
        in_cps[i].wait()
        for c in range(_CHUNKS):
            t0 = c * tc

            @plsc.parallel_loop(t0, t0 + tc, step=_L, unroll=8)
            def step(t):
                idx = codes_v[i, pl.ds(t, _L)]
                ch_v[i, pl.ds(t, _L)] = plsc.load_gather(nn_v, [idx])
                base = idx * _D
                for d in range(_D):
                    lat_v[i, d, pl.ds(t, _L)] = plsc.load_gather(w_v, [base + d])

            out_cps.append(pltpu.async_copy(
                lat_v.at[i, :, pl.ds(t0, tc)],
                lat_hbm.at[b0 + i, :, pl.ds(t0, tc)], sem_o))
            out_cps.append(pltpu.async_copy(
                ch_v.at[i, pl.ds(t0, tc)],
                ch_hbm.at[b0 + i, pl.ds(t0, tc)], sem_o))
    for cp in out_cps:
        cp.wait()


def kernel(codes, code_embed_weight):
    codes = codes.astype(jnp.int32)
    w = code_embed_weight.astype(jnp.float32)
    nn = _nn_call(w)
    latents, codes_hat = _sc_gather(w.reshape(_K * _D), nn, codes)
    return latents, codes_hat


# speedup vs baseline: 1.0335x; 1.0335x over previous
import functools

import jax
import jax.numpy as jnp
from jax import lax
from jax.experimental import pallas as pl
from jax.experimental.pallas import tpu as pltpu
from jax.experimental.pallas import tpu_sc as plsc

_B, _T, _K, _D = 64, 4096, 512, 8
_NC, _NS, _L = 2, 16, 16
_NW = _NC * _NS
_BPW = _B // _NW
_CHUNKS = 4


def _nn_body(w_ref, nn_ref):
    w = w_ref[...]
    w2_col = jnp.sum(w * w, axis=1, keepdims=True)
    ones = jnp.ones((1, _D), jnp.float32)
    w2_row = lax.dot_general(ones, w * w,
                             (((1,), (1,)), ((), ())),
                             preferred_element_type=jnp.float32)
    g = lax.dot_general(w, w, (((1,), (1,)), ((), ())),
                        preferred_element_type=jnp.float32)
    d2 = jnp.maximum(w2_col - 2.0 * g + w2_row, 0.0)
    dist = jnp.sqrt(d2)
    mn = jnp.min(dist, axis=1, keepdims=True)
    col = lax.broadcasted_iota(jnp.int32, (_K, _K), 1)
    nn = jnp.min(jnp.where(dist == mn, col, _K), axis=1, keepdims=True)
    nn_ref[...] = nn


_nn_call = pl.pallas_call(
    _nn_body,
    out_shape=jax.ShapeDtypeStruct((_K, 1), jnp.int32),
)


_sc_mesh = plsc.VectorSubcoreMesh(core_axis_name="c", subcore_axis_name="s")


@functools.partial(
    pl.kernel,
    mesh=_sc_mesh,
    compiler_params=pltpu.CompilerParams(needs_layout_passes=False,
                                         disable_bounds_checks=True,
                                         skip_device_barrier=True),
    out_type=[
        jax.ShapeDtypeStruct((_B, _D, _T), jnp.float32),
        jax.ShapeDtypeStruct((_B, _T), jnp.int32),
    ],
    scratch_types=[
        pltpu.VMEM((_K * _D,), jnp.float32),
        pltpu.VMEM((_K,), jnp.int32),
        pltpu.VMEM((_BPW, _T), jnp.int32),
        pltpu.VMEM((_BPW, _D, _T), jnp.float32),
        pltpu.VMEM((_BPW, _T), jnp.int32),
        pltpu.SemaphoreType.DMA,
        pltpu.SemaphoreType.DMA,
        pltpu.SemaphoreType.DMA,
    ],
)
def _sc_gather(w_hbm, nn_hbm, codes_hbm, lat_hbm, ch_hbm,
               w_v, nn_v, codes_v, lat_v, ch_v, sem_c0, sem_c1, sem_o):
    wid = lax.axis_index("s") * _NC + lax.axis_index("c")
    b0 = wid * _BPW
    sems = [sem_c0, sem_c1]
    in_cps = [pltpu.async_copy(codes_hbm.at[b0 + i], codes_v.at[i], sems[i])
              for i in range(_BPW)]
    pltpu.sync_copy(w_hbm, w_v)
    pltpu.sync_copy(nn_hbm, nn_v)
    out_cps = []
    for i in range(_BPW):
        in_cps[i].wait()

        @plsc.parallel_loop(0, _T, step=_L, unroll=8)
        def step(t):
            idx = codes_v[i, pl.ds(t, _L)]
            ch_v[i, pl.ds(t, _L)] = plsc.load_gather(nn_v, [idx])
            base = idx * _D
            for d in range(_D):
                lat_v[i, d, pl.ds(t, _L)] = plsc.load_gather(w_v, [base + d])

        out_cps.append(pltpu.async_copy(lat_v.at[i], lat_hbm.at[b0 + i], sem_o))
        out_cps.append(pltpu.async_copy(ch_v.at[i], ch_hbm.at[b0 + i], sem_o))
    for cp in out_cps:
        cp.wait()


def kernel(codes, code_embed_weight):
    codes = codes.astype(jnp.int32)
    w = code_embed_weight.astype(jnp.float32)
    nn = _nn_call(w).reshape(_K)
    latents, codes_hat = _sc_gather(w.reshape(_K * _D), nn, codes)
    return latents, codes_hat


# speedup vs baseline: 1.0339x; 1.0004x over previous
import functools

import jax
import jax.numpy as jnp
from jax import lax
from jax.experimental import pallas as pl
from jax.experimental.pallas import tpu as pltpu
from jax.experimental.pallas import tpu_sc as plsc

_B, _T, _K, _D = 64, 4096, 512, 8
_NC, _NS, _L = 2, 16, 16
_NW = _NC * _NS
_BPW = _B // _NW
_CHUNKS = 4


def _nn_body(w_ref, nn_ref):
    w = w_ref[...]
    w2_col = jnp.sum(w * w, axis=1, keepdims=True)
    ones = jnp.ones((1, _D), jnp.float32)
    w2_row = lax.dot_general(ones, w * w,
                             (((1,), (1,)), ((), ())),
                             preferred_element_type=jnp.float32)
    g = lax.dot_general(w, w, (((1,), (1,)), ((), ())),
                        preferred_element_type=jnp.float32)
    d2 = jnp.maximum(w2_col - 2.0 * g + w2_row, 0.0)
    dist = jnp.sqrt(d2)
    mn = jnp.min(dist, axis=1, keepdims=True)
    col = lax.broadcasted_iota(jnp.int32, (_K, _K), 1)
    nn = jnp.min(jnp.where(dist == mn, col, _K), axis=1, keepdims=True)
    nn_ref[...] = nn


_nn_call = pl.pallas_call(
    _nn_body,
    out_shape=jax.ShapeDtypeStruct((_K, 1), jnp.int32),
)


_sc_mesh = plsc.VectorSubcoreMesh(core_axis_name="c", subcore_axis_name="s")


@functools.partial(
    pl.kernel,
    mesh=_sc_mesh,
    compiler_params=pltpu.CompilerParams(needs_layout_passes=False,
                                         disable_bounds_checks=True,
                                         skip_device_barrier=True),
    out_type=[
        jax.ShapeDtypeStruct((_B, _D, _T), jnp.float32),
        jax.ShapeDtypeStruct((_B, _T), jnp.int32),
    ],
    scratch_types=[
        pltpu.VMEM((_K * _D,), jnp.float32),
        pltpu.VMEM((_K,), jnp.int32),
        pltpu.VMEM((_BPW, _T), jnp.int32),
        pltpu.VMEM((_BPW, _D, _T), jnp.float32),
        pltpu.VMEM((_BPW, _T), jnp.int32),
        pltpu.SemaphoreType.DMA,
        pltpu.SemaphoreType.DMA,
        pltpu.SemaphoreType.DMA,
    ],
)
def _sc_gather(w_hbm, nn_hbm, codes_hbm, lat_hbm, ch_hbm,
               w_v, nn_v, codes_v, lat_v, ch_v, sem_c0, sem_c1, sem_o):
    wid = lax.axis_index("s") * _NC + lax.axis_index("c")
    b0 = wid * _BPW
    sems = [sem_c0, sem_c1]
    in_cps = [pltpu.async_copy(codes_hbm.at[b0 + i], codes_v.at[i], sems[i])
              for i in range(_BPW)]
    pltpu.sync_copy(w_hbm, w_v)
    pltpu.sync_copy(nn_hbm, nn_v)
    out_cps = []
    for i in range(_BPW):
        with jax.named_scope(f"wait_in{i}"):
            in_cps[i].wait()

        scope = jax.named_scope(f"gather{i}")
        scope.__enter__()

        @plsc.parallel_loop(0, _T, step=_L, unroll=8)
        def step(t):
            idx = codes_v[i, pl.ds(t, _L)]
            ch_v[i, pl.ds(t, _L)] = plsc.load_gather(nn_v, [idx])
            base = idx * _D
            for d in range(_D):
                lat_v[i, d, pl.ds(t, _L)] = plsc.load_gather(w_v, [base + d])

        scope.__exit__(None, None, None)
        with jax.named_scope(f"issue_out{i}"):
            out_cps.append(pltpu.async_copy(lat_v.at[i], lat_hbm.at[b0 + i], sem_o))
            out_cps.append(pltpu.async_copy(ch_v.at[i], ch_hbm.at[b0 + i], sem_o))
    with jax.named_scope("drain"):
        for cp in out_cps:
            cp.wait()


def kernel(codes, code_embed_weight):
    codes = codes.astype(jnp.int32)
    w = code_embed_weight.astype(jnp.float32)
    nn = _nn_call(w).reshape(_K)
    latents, codes_hat = _sc_gather(w.reshape(_K * _D), nn, codes)
    return latents, codes_hat


# speedup vs baseline: 1.1843x; 1.1455x over previous
import functools

import jax
import jax.numpy as jnp
from jax import lax
from jax.experimental import pallas as pl
from jax.experimental.pallas import tpu as pltpu
from jax.experimental.pallas import tpu_sc as plsc

_B, _T, _K, _D = 64, 4096, 512, 8
_NC, _NS, _L = 2, 16, 16
_NW = _NC * _NS
_BPW = _B // _NW
_CHUNKS = 4


def _nn_body(w_ref, nn_ref):
    w = w_ref[...]
    w2_col = jnp.sum(w * w, axis=1, keepdims=True)
    ones = jnp.ones((1, _D), jnp.float32)
    w2_row = lax.dot_general(ones, w * w,
                             (((1,), (1,)), ((), ())),
                             preferred_element_type=jnp.float32)
    g = lax.dot_general(w, w, (((1,), (1,)), ((), ())),
                        preferred_element_type=jnp.float32)
    d2 = jnp.maximum(w2_col - 2.0 * g + w2_row, 0.0)
    dist = jnp.sqrt(d2)
    mn = jnp.min(dist, axis=1, keepdims=True)
    col = lax.broadcasted_iota(jnp.int32, (_K, _K), 1)
    nn = jnp.min(jnp.where(dist == mn, col, _K), axis=1, keepdims=True)
    nn_ref[...] = nn


_nn_call = pl.pallas_call(
    _nn_body,
    out_shape=jax.ShapeDtypeStruct((_K, 1), jnp.int32),
)


_sc_mesh = plsc.VectorSubcoreMesh(core_axis_name="c", subcore_axis_name="s")


@functools.partial(
    pl.kernel,
    mesh=_sc_mesh,
    compiler_params=pltpu.CompilerParams(needs_layout_passes=False,
                                         disable_bounds_checks=True,
                                         skip_device_barrier=True),
    out_type=[
        jax.ShapeDtypeStruct((_B, _D, _T), jnp.float32),
        jax.ShapeDtypeStruct((_B, _T), jnp.int32),
    ],
    scratch_types=[
        pltpu.VMEM((_K * _D,), jnp.float32),
        pltpu.VMEM((_K,), jnp.int32),
        pltpu.VMEM((_BPW, _T), jnp.int32),
        pltpu.VMEM((_BPW, _D, _T), jnp.float32),
        pltpu.VMEM((_BPW, _T), jnp.int32),
        pltpu.SemaphoreType.DMA,
        pltpu.SemaphoreType.DMA,
        pltpu.SemaphoreType.DMA,
    ],
)
def _sc_gather(w_hbm, nn_hbm, codes_hbm, lat_hbm, ch_hbm,
               w_v, nn_v, codes_v, lat_v, ch_v, sem_c0, sem_c1, sem_o):
    wid = lax.axis_index("s") * _NC + lax.axis_index("c")
    b0 = wid * _BPW
    sems = [sem_c0, sem_c1]
    in_cps = [pltpu.async_copy(codes_hbm.at[b0 + i], codes_v.at[i], sems[i])
              for i in range(_BPW)]
    pltpu.sync_copy(w_hbm, w_v)
    pltpu.sync_copy(nn_hbm, nn_v)
    out_cps = []
    for i in range(_BPW):
        in_cps[i].wait()

        @plsc.parallel_loop(0, _T, step=_L, unroll=8)
        def step(t):
            idx = codes_v[i, pl.ds(t, _L)]
            ch_v[i, pl.ds(t, _L)] = plsc.load_gather(nn_v, [idx])
            for d in range(_D):
                lat_v[i, d, pl.ds(t, _L)] = plsc.load_gather(w_v, [idx + d * _K])

        out_cps.append(pltpu.async_copy(lat_v.at[i], lat_hbm.at[b0 + i], sem_o))
        out_cps.append(pltpu.async_copy(ch_v.at[i], ch_hbm.at[b0 + i], sem_o))
    for cp in out_cps:
        cp.wait()


def kernel(codes, code_embed_weight):
    codes = codes.astype(jnp.int32)
    w = code_embed_weight.astype(jnp.float32)
    nn = _nn_call(w).reshape(_K)
    latents, codes_hat = _sc_gather(w.T.reshape(_K * _D), nn, codes)
    return latents, codes_hat


# speedup vs baseline: 1.1905x; 1.0052x over previous
import functools

import jax
import jax.numpy as jnp
from jax import lax
from jax.experimental import pallas as pl
from jax.experimental.pallas import tpu as pltpu
from jax.experimental.pallas import tpu_sc as plsc

_B, _T, _K, _D = 64, 4096, 512, 8
_NC, _NS, _L = 2, 16, 16
_NW = _NC * _NS
_BPW = _B // _NW
_CHUNKS = 4


def _nn_body(w_ref, nn_ref):
    w = w_ref[...]
    w2_col = jnp.sum(w * w, axis=1, keepdims=True)
    ones = jnp.ones((1, _D), jnp.float32)
    w2_row = lax.dot_general(ones, w * w,
                             (((1,), (1,)), ((), ())),
                             preferred_element_type=jnp.float32)
    g = lax.dot_general(w, w, (((1,), (1,)), ((), ())),
                        preferred_element_type=jnp.float32)
    d2 = jnp.maximum(w2_col - 2.0 * g + w2_row, 0.0)
    dist = jnp.sqrt(d2)
    mn = jnp.min(dist, axis=1, keepdims=True)
    col = lax.broadcasted_iota(jnp.int32, (_K, _K), 1)
    nn = jnp.min(jnp.where(dist == mn, col, _K), axis=1, keepdims=True)
    nn_ref[...] = nn


_nn_call = pl.pallas_call(
    _nn_body,
    out_shape=jax.ShapeDtypeStruct((_K, 1), jnp.int32),
)


_sc_mesh = plsc.VectorSubcoreMesh(core_axis_name="c", subcore_axis_name="s")


@functools.partial(
    pl.kernel,
    mesh=_sc_mesh,
    compiler_params=pltpu.CompilerParams(needs_layout_passes=False,
                                         disable_bounds_checks=True,
                                         skip_device_barrier=True),
    out_type=[
        jax.ShapeDtypeStruct((_B, _D, _T), jnp.float32),
        jax.ShapeDtypeStruct((_B, _T), jnp.int32),
    ],
    scratch_types=[
        pltpu.VMEM((_K * _D,), jnp.float32),
        pltpu.VMEM((_K,), jnp.int32),
        pltpu.VMEM((_BPW, _T), jnp.int32),
        pltpu.VMEM((_BPW, _D, _T), jnp.float32),
        pltpu.VMEM((_BPW, _T), jnp.int32),
        pltpu.SemaphoreType.DMA,
        pltpu.SemaphoreType.DMA,
        pltpu.SemaphoreType.DMA,
        pltpu.SemaphoreType.DMA,
    ],
)
def _sc_gather(w_hbm, nn_hbm, codes_hbm, lat_hbm, ch_hbm,
               w_v, nn_v, codes_v, lat_v, ch_v, sem_w, sem_c0, sem_c1, sem_o):
    wid = lax.axis_index("s") * _NC + lax.axis_index("c")
    b0 = wid * _BPW
    w_cp = pltpu.async_copy(w_hbm, w_v, sem_w)
    nn_cp = pltpu.async_copy(nn_hbm, nn_v, sem_w)
    sems = [sem_c0, sem_c1]
    in_cps = [pltpu.async_copy(codes_hbm.at[b0 + i], codes_v.at[i], sems[i])
              for i in range(_BPW)]
    w_cp.wait()
    nn_cp.wait()
    out_cps = []
    for i in range(_BPW):
        in_cps[i].wait()

        @plsc.parallel_loop(0, _T, step=_L, unroll=8)
        def step(t):
            idx = codes_v[i, pl.ds(t, _L)]
            ch_v[i, pl.ds(t, _L)] = plsc.load_gather(nn_v, [idx])
            for d in range(_D):
                lat_v[i, d, pl.ds(t, _L)] = plsc.load_gather(w_v, [idx + d * _K])

        out_cps.append(pltpu.async_copy(lat_v.at[i], lat_hbm.at[b0 + i], sem_o))
        out_cps.append(pltpu.async_copy(ch_v.at[i], ch_hbm.at[b0 + i], sem_o))
    for cp in out_cps:
        cp.wait()


def kernel(codes, code_embed_weight):
    codes = codes.astype(jnp.int32)
    w = code_embed_weight.astype(jnp.float32)
    nn = _nn_call(w).reshape(_K)
    latents, codes_hat = _sc_gather(w.T.reshape(_K * _D), nn, codes)
    return latents, codes_hat


# speedup vs baseline: 1.2207x; 1.0254x over previous
import functools

import jax
import jax.numpy as jnp
from jax import lax
from jax.experimental import pallas as pl
from jax.experimental.pallas import tpu as pltpu
from jax.experimental.pallas import tpu_sc as plsc

_B, _T, _K, _D = 64, 4096, 512, 8
_NC, _NS, _L = 2, 16, 16
_NW = _NC * _NS
_BPW = _B // _NW
_CHUNKS = 4


def _nn_body(w_ref, nn_ref):
    w = w_ref[...]
    w2_col = jnp.sum(w * w, axis=1, keepdims=True)
    ones = jnp.ones((1, _D), jnp.float32)
    w2_row = lax.dot_general(ones, w * w,
                             (((1,), (1,)), ((), ())),
                             preferred_element_type=jnp.float32)
    g = lax.dot_general(w, w, (((1,), (1,)), ((), ())),
                        preferred_element_type=jnp.float32)
    d2 = jnp.maximum(w2_col - 2.0 * g + w2_row, 0.0)
    dist = jnp.sqrt(d2)
    mn = jnp.min(dist, axis=1, keepdims=True)
    col = lax.broadcasted_iota(jnp.int32, (_K, _K), 1)
    nn = jnp.min(jnp.where(dist == mn, col, _K), axis=1)
    nn_ref[...] = nn


_nn_call = pl.pallas_call(
    _nn_body,
    out_shape=jax.ShapeDtypeStruct((_K,), jnp.int32),
)


_sc_mesh = plsc.VectorSubcoreMesh(core_axis_name="c", subcore_axis_name="s")


@functools.partial(
    pl.kernel,
    mesh=_sc_mesh,
    compiler_params=pltpu.CompilerParams(needs_layout_passes=False,
                                         disable_bounds_checks=True,
                                         skip_device_barrier=True),
    out_type=[
        jax.ShapeDtypeStruct((_B, _D, _T), jnp.float32),
        jax.ShapeDtypeStruct((_B, _T), jnp.int32),
    ],
    scratch_types=[
        pltpu.VMEM((_K * _D,), jnp.float32),
        pltpu.VMEM((_K * 9,), jnp.float32),
        pltpu.VMEM((_K,), jnp.int32),
        pltpu.VMEM((_BPW, _T), jnp.int32),
        pltpu.VMEM((_BPW, _D, _T), jnp.float32),
        pltpu.VMEM((_BPW, _T), jnp.int32),
        pltpu.SemaphoreType.DMA,
        pltpu.SemaphoreType.DMA,
        pltpu.SemaphoreType.DMA,
        pltpu.SemaphoreType.DMA,
    ],
)
def _sc_gather(w_hbm, nn_hbm, codes_hbm, lat_hbm, ch_hbm,
               w_v, wp_v, nn_v, codes_v, lat_v, ch_v, sem_w, sem_c0, sem_c1, sem_o):
    wid = lax.axis_index("s") * _NC + lax.axis_index("c")
    b0 = wid * _BPW
    w_cp = pltpu.async_copy(w_hbm, w_v, sem_w)
    nn_cp = pltpu.async_copy(nn_hbm, nn_v, sem_w)
    sems = [sem_c0, sem_c1]
    in_cps = [pltpu.async_copy(codes_hbm.at[b0 + i], codes_v.at[i], sems[i])
              for i in range(_BPW)]
    w_cp.wait()
    nn_cp.wait()

    pat = 9 * lax.shift_right_logical(lax.iota(jnp.int32, _L), 3) \
        + lax.bitwise_and(lax.iota(jnp.int32, _L), 7)

    @plsc.parallel_loop(0, _K * _D // _L, step=1, unroll=8)
    def repack(c):
        vals = w_v[pl.ds(c * _L, _L)]
        plsc.store_scatter(wp_v, [pat + 18 * c], vals)

    out_cps = []
    for i in range(_BPW):
        in_cps[i].wait()

        @plsc.parallel_loop(0, _T, step=_L, unroll=8)
        def step(t):
            idx = codes_v[i, pl.ds(t, _L)]
            ch_v[i, pl.ds(t, _L)] = plsc.load_gather(nn_v, [idx])
            base = idx * 9
            for d in range(_D):
                lat_v[i, d, pl.ds(t, _L)] = plsc.load_gather(wp_v, [base + d])

        out_cps.append(pltpu.async_copy(lat_v.at[i], lat_hbm.at[b0 + i], sem_o))
        out_cps.append(pltpu.async_copy(ch_v.at[i], ch_hbm.at[b0 + i], sem_o))
    for cp in out_cps:
        cp.wait()


def kernel(codes, code_embed_weight):
    codes = codes.astype(jnp.int32)
    w = code_embed_weight.astype(jnp.float32)
    nn = _nn_call(w)
    latents, codes_hat = _sc_gather(w.reshape(_K * _D), nn, codes)
    return latents, codes_hat


# speedup vs baseline: 1.2221x; 1.0012x over previous
import functools

import jax
import jax.numpy as jnp
from jax import lax
from jax.experimental import pallas as pl
from jax.experimental.pallas import tpu as pltpu
from jax.experimental.pallas import tpu_sc as plsc

_B, _T, _K, _D = 64, 4096, 512, 8
_NC, _NS, _L = 2, 16, 16
_NW = _NC * _NS
_BPW = _B // _NW
_CHUNKS = 4


def _nn_body(w_ref, nn_ref):
    w = w_ref[...]
    w2_col = jnp.sum(w * w, axis=1, keepdims=True)
    ones = jnp.ones((1, _D), jnp.float32)
    w2_row = lax.dot_general(ones, w * w,
                             (((1,), (1,)), ((), ())),
                             preferred_element_type=jnp.float32)
    g = lax.dot_general(w, w, (((1,), (1,)), ((), ())),
                        preferred_element_type=jnp.float32)
    d2 = jnp.maximum(w2_col - 2.0 * g + w2_row, 0.0)
    dist = jnp.sqrt(d2)
    mn = jnp.min(dist, axis=1, keepdims=True)
    col = lax.broadcasted_iota(jnp.int32, (_K, _K), 1)
    nn = jnp.min(jnp.where(dist == mn, col, _K), axis=1)
    nn_ref[...] = nn


_nn_call = pl.pallas_call(
    _nn_body,
    out_shape=jax.ShapeDtypeStruct((_K,), jnp.int32),
)


_sc_mesh = plsc.VectorSubcoreMesh(core_axis_name="c", subcore_axis_name="s")


@functools.partial(
    pl.kernel,
    mesh=_sc_mesh,
    compiler_params=pltpu.CompilerParams(needs_layout_passes=False,
                                         disable_bounds_checks=True,
                                         skip_device_barrier=True),
    out_type=[
        jax.ShapeDtypeStruct((_B, _D, _T), jnp.float32),
        jax.ShapeDtypeStruct((_B, _T), jnp.int32),
    ],
    scratch_types=[
        pltpu.VMEM((32, 128), jnp.float32),
        pltpu.VMEM((_K * 9,), jnp.float32),
        pltpu.VMEM((_K,), jnp.int32),
        pltpu.VMEM((_BPW, _T), jnp.int32),
        pltpu.VMEM((_BPW, _D, _T), jnp.float32),
        pltpu.VMEM((_BPW, _T), jnp.int32),
        pltpu.SemaphoreType.DMA,
        pltpu.SemaphoreType.DMA,
        pltpu.SemaphoreType.DMA,
        pltpu.SemaphoreType.DMA,
    ],
)
def _sc_gather(w_hbm, nn_hbm, codes_hbm, lat_hbm, ch_hbm,
               w_v, wp_v, nn_v, codes_v, lat_v, ch_v, sem_w, sem_c0, sem_c1, sem_o):
    wid = lax.axis_index("s") * _NC + lax.axis_index("c")
    b0 = wid * _BPW
    w_cp = pltpu.async_copy(w_hbm, w_v, sem_w)
    nn_cp = pltpu.async_copy(nn_hbm, nn_v, sem_w)
    sems = [sem_c0, sem_c1]
    in_cps = [pltpu.async_copy(codes_hbm.at[b0 + i], codes_v.at[i], sems[i])
              for i in range(_BPW)]
    w_cp.wait()
    nn_cp.wait()

    iota = lax.iota(jnp.int32, _L)
    hi, lo = lax.shift_right_logical(iota, 3), lax.bitwise_and(iota, 7)
    pat = 9 * hi + lo

    @plsc.parallel_loop(0, _K * _D // _L, step=1, unroll=8)
    def repack(c):
        vals = w_v[c >> 3, pl.ds((c & 7) * _L, _L)]
        plsc.store_scatter(wp_v, [pat + 18 * c], vals)

    out_cps = []
    for i in range(_BPW):
        in_cps[i].wait()

        @plsc.parallel_loop(0, _T, step=_L, unroll=8)
        def step(t):
            idx = codes_v[i, pl.ds(t, _L)]
            ch_v[i, pl.ds(t, _L)] = plsc.load_gather(nn_v, [idx])
            base = idx * 9
            for d in range(_D):
                lat_v[i, d, pl.ds(t, _L)] = plsc.load_gather(wp_v, [base + d])

        out_cps.append(pltpu.async_copy(lat_v.at[i], lat_hbm.at[b0 + i], sem_o))
        out_cps.append(pltpu.async_copy(ch_v.at[i], ch_hbm.at[b0 + i], sem_o))
    for cp in out_cps:
        cp.wait()


def kernel(codes, code_embed_weight):
    codes = codes.astype(jnp.int32)
    w = code_embed_weight.astype(jnp.float32)
    nn = _nn_call(w)
    latents, codes_hat = _sc_gather(w.reshape(32, 128), nn, codes)
    return latents, codes_hat


# speedup vs baseline: 1.3139x; 1.0751x over previous
import functools

import jax
import jax.numpy as jnp
from jax import lax
from jax.experimental import pallas as pl
from jax.experimental.pallas import tpu as pltpu
from jax.experimental.pallas import tpu_sc as plsc

_B, _T, _K, _D = 64, 4096, 512, 8
_NC, _NS, _L = 2, 16, 16
_NW = _NC * _NS
_BPW = _B // _NW
_CHUNKS = 4


def _nn_body(wt_ref, nn_ref):
    wt = wt_ref[...]
    w2_row = jnp.sum(wt * wt, axis=0, keepdims=True)
    ones = jnp.ones((_D, 1), jnp.float32)
    w2_col = lax.dot_general(wt * wt, ones,
                             (((0,), (0,)), ((), ())),
                             preferred_element_type=jnp.float32)
    g = lax.dot_general(wt, wt, (((0,), (0,)), ((), ())),
                        preferred_element_type=jnp.float32)
    d2 = jnp.maximum(w2_col - 2.0 * g + w2_row, 0.0)
    dist = jnp.sqrt(d2)
    mn = jnp.min(dist, axis=1, keepdims=True)
    col = lax.broadcasted_iota(jnp.int32, (_K, _K), 1)
    nn = jnp.min(jnp.where(dist == mn, col, _K), axis=1)
    nn_ref[...] = nn


_nn_call = pl.pallas_call(
    _nn_body,
    out_shape=jax.ShapeDtypeStruct((_K,), jnp.int32),
)


_sc_mesh = plsc.VectorSubcoreMesh(core_axis_name="c", subcore_axis_name="s")


@functools.partial(
    pl.kernel,
    mesh=_sc_mesh,
    compiler_params=pltpu.CompilerParams(needs_layout_passes=False,
                                         disable_bounds_checks=True,
                                         skip_device_barrier=True),
    out_type=[
        jax.ShapeDtypeStruct((_B, _D, _T), jnp.float32),
        jax.ShapeDtypeStruct((_B, _T), jnp.int32),
    ],
    scratch_types=[
        pltpu.VMEM((32, 128), jnp.float32),
        pltpu.VMEM((_K * 9,), jnp.float32),
        pltpu.VMEM((_K,), jnp.int32),
        pltpu.VMEM((_BPW, _T), jnp.int32),
        pltpu.VMEM((_BPW, _D, _T), jnp.float32),
        pltpu.VMEM((_BPW, _T), jnp.int32),
        pltpu.SemaphoreType.DMA,
        pltpu.SemaphoreType.DMA,
        pltpu.SemaphoreType.DMA,
        pltpu.SemaphoreType.DMA,
    ],
)
def _sc_gather(w_hbm, nn_hbm, codes_hbm, lat_hbm, ch_hbm,
               w_v, wp_v, nn_v, codes_v, lat_v, ch_v, sem_w, sem_c0, sem_c1, sem_o):
    wid = lax.axis_index("s") * _NC + lax.axis_index("c")
    b0 = wid * _BPW
    w_cp = pltpu.async_copy(w_hbm, w_v, sem_w)
    nn_cp = pltpu.async_copy(nn_hbm, nn_v, sem_w)
    sems = [sem_c0, sem_c1]
    in_cps = [pltpu.async_copy(codes_hbm.at[b0 + i], codes_v.at[i], sems[i])
              for i in range(_BPW)]
    w_cp.wait()
    nn_cp.wait()

    pat = 9 * lax.iota(jnp.int32, _L)

    @plsc.parallel_loop(0, _K * _D // _L, step=1, unroll=8)
    def repack(c):
        vals = w_v[c >> 3, pl.ds((c & 7) * _L, _L)]
        plsc.store_scatter(wp_v, [pat + (144 * (c & 31) + (c >> 5))], vals)

    out_cps = []
    for i in range(_BPW):
        in_cps[i].wait()

        @plsc.parallel_loop(0, _T, step=_L, unroll=8)
        def step(t):
            idx = codes_v[i, pl.ds(t, _L)]
            ch_v[i, pl.ds(t, _L)] = plsc.load_gather(nn_v, [idx])
            base = idx * 9
            for d in range(_D):
                lat_v[i, d, pl.ds(t, _L)] = plsc.load_gather(wp_v, [base + d])

        out_cps.append(pltpu.async_copy(lat_v.at[i], lat_hbm.at[b0 + i], sem_o))
        out_cps.append(pltpu.async_copy(ch_v.at[i], ch_hbm.at[b0 + i], sem_o))
    for cp in out_cps:
        cp.wait()


def kernel(codes, code_embed_weight):
    codes = codes.astype(jnp.int32)
    w = code_embed_weight.astype(jnp.float32)
    wt = w.T
    nn = _nn_call(wt)
    latents, codes_hat = _sc_gather(wt.reshape(32, 128), nn, codes)
    return latents, codes_hat


# speedup vs baseline: 1.3174x; 1.0026x over previous
import functools

import jax
import jax.numpy as jnp
from jax import lax
from jax.experimental import pallas as pl
from jax.experimental.pallas import tpu as pltpu
from jax.experimental.pallas import tpu_sc as plsc

_B, _T, _K, _D = 64, 4096, 512, 8
_NC, _NS, _L = 2, 16, 16
_NW = _NC * _NS
_BPW = _B // _NW
_CHUNKS = 4


def _nn_body(wt_ref, nn_ref):
    wt = wt_ref[...]
    w2_row = jnp.sum(wt * wt, axis=0, keepdims=True)
    g = lax.dot_general(wt, wt, (((0,), (0,)), ((), ())),
                        preferred_element_type=jnp.float32)
    e = w2_row - 2.0 * g
    mn = jnp.min(e, axis=1, keepdims=True)
    col = lax.broadcasted_iota(jnp.int32, (_K, _K), 1)
    nn = jnp.min(jnp.where(e == mn, col, _K), axis=1)
    nn_ref[...] = nn


_nn_call = pl.pallas_call(
    _nn_body,
    out_shape=jax.ShapeDtypeStruct((_K,), jnp.int32),
)


_sc_mesh = plsc.VectorSubcoreMesh(core_axis_name="c", subcore_axis_name="s")


@functools.partial(
    pl.kernel,
    mesh=_sc_mesh,
    compiler_params=pltpu.CompilerParams(needs_layout_passes=False,
                                         disable_bounds_checks=True,
                                         skip_device_barrier=True),
    out_type=[
        jax.ShapeDtypeStruct((_B, _D, _T), jnp.float32),
        jax.ShapeDtypeStruct((_B, _T), jnp.int32),
    ],
    scratch_types=[
        pltpu.VMEM((32, 128), jnp.float32),
        pltpu.VMEM((_K * 9,), jnp.float32),
        pltpu.VMEM((_K,), jnp.int32),
        pltpu.VMEM((_BPW, _T), jnp.int32),
        pltpu.VMEM((_BPW, _D, _T), jnp.float32),
        pltpu.VMEM((_BPW, _T), jnp.int32),
        pltpu.SemaphoreType.DMA,
        pltpu.SemaphoreType.DMA,
        pltpu.SemaphoreType.DMA,
        pltpu.SemaphoreType.DMA,
    ],
)
def _sc_gather(w_hbm, nn_hbm, codes_hbm, lat_hbm, ch_hbm,
               w_v, wp_v, nn_v, codes_v, lat_v, ch_v, sem_w, sem_c0, sem_c1, sem_o):
    wid = lax.axis_index("s") * _NC + lax.axis_index("c")
    b0 = wid * _BPW
    w_cp = pltpu.async_copy(w_hbm, w_v, sem_w)
    nn_cp = pltpu.async_copy(nn_hbm, nn_v, sem_w)
    sems = [sem_c0, sem_c1]
    in_cps = [pltpu.async_copy(codes_hbm.at[b0 + i], codes_v.at[i], sems[i])
              for i in range(_BPW)]
    w_cp.wait()
    nn_cp.wait()

    pat = 9 * lax.iota(jnp.int32, _L)

    @plsc.parallel_loop(0, _K * _D // _L, step=1, unroll=8)
    def repack(c):
        vals = w_v[c >> 3, pl.ds((c & 7) * _L, _L)]
        plsc.store_scatter(wp_v, [pat + (144 * (c & 31) + (c >> 5))], vals)

    out_cps = []
    for i in range(_BPW):
        in_cps[i].wait()

        @plsc.parallel_loop(0, _T, step=_L, unroll=8)
        def step(t):
            idx = codes_v[i, pl.ds(t, _L)]
            ch_v[i, pl.ds(t, _L)] = plsc.load_gather(nn_v, [idx])
            base = idx * 9
            for d in range(_D):
                lat_v[i, d, pl.ds(t, _L)] = plsc.load_gather(wp_v, [base + d])

        out_cps.append(pltpu.async_copy(lat_v.at[i], lat_hbm.at[b0 + i], sem_o))
        out_cps.append(pltpu.async_copy(ch_v.at[i], ch_hbm.at[b0 + i], sem_o))
    for cp in out_cps:
        cp.wait()


def kernel(codes, code_embed_weight):
    codes = codes.astype(jnp.int32)
    w = code_embed_weight.astype(jnp.float32)
    wt = w.T
    nn = _nn_call(wt)
    latents, codes_hat = _sc_gather(wt.reshape(32, 128), nn, codes)
    return latents, codes_hat
